# Initial kernel scaffold; baseline (speedup 1.0000x reference)
import jax
import jax.numpy as jnp
from jax.experimental import pallas as pl


def kernel(x, edge_index, W1, b1, W_mu, b_mu, W_ls, b_ls):
    raise NotImplementedError("write your pallas kernel here")


# speedup vs baseline: 32.2724x; 32.2724x over previous
import functools

import jax
import jax.numpy as jnp
from jax import lax
from jax.experimental import pallas as pl
from jax.experimental.pallas import tpu as pltpu
from jax.experimental.pallas import tpu_sc as plsc

N = 10000
E = 320000
D_IN = 128
D_HID = 32
D_OUT = 16

NC = 2
NS = 16
NW = NC * NS

CH = 128
CPT = -(-E // (NW * CH))
EP = NW * CPT * CH

NP = 10240
RPS = NP // NS

_mesh = plsc.VectorSubcoreMesh(core_axis_name="c", subcore_axis_name="s")


@functools.partial(
    pl.kernel,
    out_type=jax.ShapeDtypeStruct((NC, NP), jnp.float32),
    mesh=_mesh,
    scratch_types=[
        pltpu.VMEM((CPT, CH), jnp.int32),
        pltpu.VMEM((CH,), jnp.float32),
        pltpu.VMEM_SHARED((NP,), jnp.float32),
        pltpu.SemaphoreType.DMA,
    ],
)
def _sc_deg(z_hbm, dst_hbm, out_hbm, dst_v, ones_v, acc_sh, sem):
    c = lax.axis_index("c")
    s = lax.axis_index("s")
    wid = c * NS + s
    row0 = s * RPS
    pltpu.sync_copy(dst_hbm.at[wid], dst_v)
    for k in range(CH // 16):
        ones_v[pl.ds(k * 16, 16)] = jnp.full((16,), 1.0, jnp.float32)
    pltpu.sync_copy(z_hbm.at[pl.ds(row0, RPS)], acc_sh.at[pl.ds(row0, RPS)])
    plsc.subcore_barrier()

    def body(j, carry):
        pltpu.sync_copy(ones_v, acc_sh.at[dst_v.at[j]], add=True)
        return carry

    lax.fori_loop(0, CPT, body, 0)
    plsc.subcore_barrier()
    pltpu.sync_copy(acc_sh.at[pl.ds(row0, RPS)], out_hbm.at[c, pl.ds(row0, RPS)])


@functools.partial(
    pl.kernel,
    out_type=jax.ShapeDtypeStruct((NC, NP, D_HID), jnp.float32),
    mesh=_mesh,
    scratch_types=[
        pltpu.VMEM((CPT, CH), jnp.int32),
        pltpu.VMEM((CPT, CH), jnp.int32),
        pltpu.VMEM((CH, D_HID), jnp.float32),
        pltpu.VMEM_SHARED((NP, D_HID), jnp.float32),
        pltpu.SemaphoreType.DMA,
    ],
    compiler_params=pltpu.CompilerParams(use_tc_tiling_on_sc=False),
)
def _sc_pass(g_hbm, src_hbm, dst_hbm, out_hbm, src_v, dst_v, rows_v, acc_sh, sem):
    c = lax.axis_index("c")
    s = lax.axis_index("s")
    wid = c * NS + s
    row0 = s * RPS
    pltpu.sync_copy(src_hbm.at[wid], src_v)
    pltpu.sync_copy(dst_hbm.at[wid], dst_v)
    pltpu.sync_copy(g_hbm.at[pl.ds(row0, RPS)], acc_sh.at[pl.ds(row0, RPS)])
    plsc.subcore_barrier()

    def body(j, carry):
        pltpu.async_copy(g_hbm.at[src_v.at[j]], rows_v, sem).wait()
        pltpu.sync_copy(rows_v, acc_sh.at[dst_v.at[j]], add=True)
        return carry

    lax.fori_loop(0, CPT, body, 0)
    plsc.subcore_barrier()
    pltpu.sync_copy(acc_sh.at[pl.ds(row0, RPS)], out_hbm.at[c, pl.ds(row0, RPS)])


def _mm1_body(x_ref, w_ref, deg_ref, g_ref, dinv_ref):
    dinv = lax.rsqrt(deg_ref[0] + deg_ref[1] + 1.0)
    h = jnp.dot(x_ref[...], w_ref[...], preferred_element_type=jnp.float32)
    g_ref[...] = h * dinv
    dinv_ref[...] = dinv


_mm1 = pl.pallas_call(
    _mm1_body,
    out_shape=(
        jax.ShapeDtypeStruct((NP, D_HID), jnp.float32),
        jax.ShapeDtypeStruct((NP, 1), jnp.float32),
    ),
)


def _mid_body(b_ref, g1_ref, dinv_ref, bias_ref, g2_ref):
    agg = dinv_ref[...] * (b_ref[0] + b_ref[1] - g1_ref[...])
    h = jnp.maximum(agg + bias_ref[...], 0.0)
    g2_ref[...] = dinv_ref[...] * h


_mid = pl.pallas_call(
    _mid_body,
    out_shape=jax.ShapeDtypeStruct((NP, D_HID), jnp.float32),
)


def _fin_body(b_ref, g2_ref, dinv_ref, wmu_ref, bmu_ref, wls_ref, bls_ref,
              mu_ref, ls_ref):
    a = dinv_ref[...] * (b_ref[0] + b_ref[1] - g2_ref[...])
    mu_ref[...] = jnp.dot(a, wmu_ref[...], preferred_element_type=jnp.float32) + bmu_ref[...]
    ls_ref[...] = jnp.dot(a, wls_ref[...], preferred_element_type=jnp.float32) + bls_ref[...]


_fin = pl.pallas_call(
    _fin_body,
    out_shape=(
        jax.ShapeDtypeStruct((NP, D_OUT), jnp.float32),
        jax.ShapeDtypeStruct((NP, D_OUT), jnp.float32),
    ),
)


def kernel(x, edge_index, W1, b1, W_mu, b_mu, W_ls, b_ls):
    src = edge_index[0]
    dst = edge_index[1]
    pad = EP - E
    srcp = jnp.concatenate([src, jnp.zeros((pad,), src.dtype)]).reshape(NW, CPT, CH)
    dstp = jnp.concatenate([dst, jnp.full((pad,), N, dst.dtype)]).reshape(NW, CPT, CH)
    xp = jnp.concatenate([x, jnp.zeros((NP - N, D_IN), x.dtype)])
    z = jnp.zeros((NP,), jnp.float32)

    deg = _sc_deg(z, dstp).reshape(NC, NP, 1)
    g1, dinv = _mm1(xp, W1, deg)
    b1p = _sc_pass(g1, srcp, dstp)
    g2 = _mid(b1p, g1, dinv, b1.reshape(1, D_HID))
    b2p = _sc_pass(g2, srcp, dstp)
    mu, ls = _fin(b2p, g2, dinv, W_mu, b_mu.reshape(1, D_OUT),
                  W_ls, b_ls.reshape(1, D_OUT))
    return mu[:N], ls[:N]


# speedup vs baseline: 40.8905x; 1.2670x over previous
import functools

import jax
import jax.numpy as jnp
from jax import lax
from jax.experimental import pallas as pl
from jax.experimental.pallas import tpu as pltpu
from jax.experimental.pallas import tpu_sc as plsc

N = 10000
E = 320000
D_IN = 128
D_HID = 32
D_OUT = 16

NC = 2
NS = 16
NW = NC * NS

CH = 128
CPT = -(-E // (NW * CH))
EP = NW * CPT * CH

NP = 10240
RPS = NP // NS

_mesh = plsc.VectorSubcoreMesh(core_axis_name="c", subcore_axis_name="s")


@functools.partial(
    pl.kernel,
    out_type=jax.ShapeDtypeStruct((NC, NP), jnp.float32),
    mesh=_mesh,
    scratch_types=[
        pltpu.VMEM((CPT, CH), jnp.int32),
        pltpu.VMEM((CH,), jnp.float32),
        pltpu.VMEM_SHARED((NP,), jnp.float32),
        pltpu.SemaphoreType.DMA,
    ],
)
def _sc_deg(z_hbm, dst_hbm, out_hbm, dst_v, ones_v, acc_sh, sem):
    c = lax.axis_index("c")
    s = lax.axis_index("s")
    wid = c * NS + s
    row0 = s * RPS
    pltpu.sync_copy(dst_hbm.at[wid], dst_v)
    for k in range(CH // 16):
        ones_v[pl.ds(k * 16, 16)] = jnp.full((16,), 1.0, jnp.float32)
    pltpu.sync_copy(z_hbm.at[pl.ds(row0, RPS)], acc_sh.at[pl.ds(row0, RPS)])
    plsc.subcore_barrier()

    def body(j, carry):
        pltpu.sync_copy(ones_v, acc_sh.at[dst_v.at[j]], add=True)
        return carry

    lax.fori_loop(0, CPT, body, 0)
    plsc.subcore_barrier()
    pltpu.sync_copy(acc_sh.at[pl.ds(row0, RPS)], out_hbm.at[c, pl.ds(row0, RPS)])


@functools.partial(
    pl.kernel,
    out_type=jax.ShapeDtypeStruct((NC, NP, D_HID), jnp.float32),
    mesh=_mesh,
    scratch_types=[
        pltpu.VMEM((CPT, CH), jnp.int32),
        pltpu.VMEM((CPT, CH), jnp.int32),
        pltpu.VMEM((CH, D_HID), jnp.float32),
        pltpu.VMEM((CH, D_HID), jnp.float32),
        pltpu.VMEM_SHARED((NP, D_HID), jnp.float32),
        pltpu.SemaphoreType.DMA,
        pltpu.SemaphoreType.DMA,
    ],
    compiler_params=pltpu.CompilerParams(use_tc_tiling_on_sc=False),
)
def _sc_pass(g_hbm, src_hbm, dst_hbm, out_hbm, src_v, dst_v, rows0, rows1,
             acc_sh, gsem, ssem):
    c = lax.axis_index("c")
    s = lax.axis_index("s")
    wid = c * NS + s
    row0 = s * RPS
    pltpu.sync_copy(src_hbm.at[wid], src_v)
    pltpu.sync_copy(dst_hbm.at[wid], dst_v)
    pltpu.sync_copy(g_hbm.at[pl.ds(row0, RPS)], acc_sh.at[pl.ds(row0, RPS)])
    plsc.subcore_barrier()

    def gwait():
        pltpu.make_async_copy(g_hbm.at[src_v.at[0]], rows0, gsem).wait()

    def swait():
        pltpu.make_async_copy(rows0, acc_sh.at[dst_v.at[0]], ssem).wait()

    pltpu.async_copy(g_hbm.at[src_v.at[0]], rows0, gsem)

    def body(k, carry):
        p = 2 * k + 1
        q = 2 * k + 2
        @pl.when(k > 0)
        def _():
            swait()
        pltpu.async_copy(g_hbm.at[src_v.at[p]], rows1, gsem)
        gwait()
        pltpu.async_copy(rows0, acc_sh.at[dst_v.at[2 * k]], ssem, add=True)
        swait()
        pltpu.async_copy(g_hbm.at[src_v.at[q]], rows0, gsem)
        gwait()
        pltpu.async_copy(rows1, acc_sh.at[dst_v.at[p]], ssem, add=True)
        return carry

    lax.fori_loop(0, (CPT - 1) // 2, body, 0)
    swait()
    gwait()
    pltpu.sync_copy(rows0, acc_sh.at[dst_v.at[CPT - 1]], add=True)
    plsc.subcore_barrier()
    pltpu.sync_copy(acc_sh.at[pl.ds(row0, RPS)], out_hbm.at[c, pl.ds(row0, RPS)])


def _mm1_body(x_ref, w_ref, deg_ref, g_ref, dinv_ref):
    dinv = lax.rsqrt(deg_ref[0] + deg_ref[1] + 1.0)
    h = jnp.dot(x_ref[...], w_ref[...], preferred_element_type=jnp.float32)
    g_ref[...] = h * dinv
    dinv_ref[...] = dinv


_mm1 = pl.pallas_call(
    _mm1_body,
    out_shape=(
        jax.ShapeDtypeStruct((NP, D_HID), jnp.float32),
        jax.ShapeDtypeStruct((NP, 1), jnp.float32),
    ),
)


def _mid_body(b_ref, g1_ref, dinv_ref, bias_ref, g2_ref):
    agg = dinv_ref[...] * (b_ref[0] + b_ref[1] - g1_ref[...])
    h = jnp.maximum(agg + bias_ref[...], 0.0)
    g2_ref[...] = dinv_ref[...] * h


_mid = pl.pallas_call(
    _mid_body,
    out_shape=jax.ShapeDtypeStruct((NP, D_HID), jnp.float32),
)


def _fin_body(b_ref, g2_ref, dinv_ref, wmu_ref, bmu_ref, wls_ref, bls_ref,
              mu_ref, ls_ref):
    a = dinv_ref[...] * (b_ref[0] + b_ref[1] - g2_ref[...])
    mu_ref[...] = jnp.dot(a, wmu_ref[...], preferred_element_type=jnp.float32) + bmu_ref[...]
    ls_ref[...] = jnp.dot(a, wls_ref[...], preferred_element_type=jnp.float32) + bls_ref[...]


_fin = pl.pallas_call(
    _fin_body,
    out_shape=(
        jax.ShapeDtypeStruct((NP, D_OUT), jnp.float32),
        jax.ShapeDtypeStruct((NP, D_OUT), jnp.float32),
    ),
)


def kernel(x, edge_index, W1, b1, W_mu, b_mu, W_ls, b_ls):
    src = edge_index[0]
    dst = edge_index[1]
    pad = EP - E
    srcp = jnp.concatenate([src, jnp.zeros((pad,), src.dtype)]).reshape(NW, CPT, CH)
    dstp = jnp.concatenate([dst, jnp.full((pad,), N, dst.dtype)]).reshape(NW, CPT, CH)
    xp = jnp.concatenate([x, jnp.zeros((NP - N, D_IN), x.dtype)])
    z = jnp.zeros((NP,), jnp.float32)

    deg = _sc_deg(z, dstp).reshape(NC, NP, 1)
    g1, dinv = _mm1(xp, W1, deg)
    b1p = _sc_pass(g1, srcp, dstp)
    g2 = _mid(b1p, g1, dinv, b1.reshape(1, D_HID))
    b2p = _sc_pass(g2, srcp, dstp)
    mu, ls = _fin(b2p, g2, dinv, W_mu, b_mu.reshape(1, D_OUT),
                  W_ls, b_ls.reshape(1, D_OUT))
    return mu[:N], ls[:N]


# speedup vs baseline: 43.6656x; 1.0679x over previous
import functools

import jax
import jax.numpy as jnp
from jax import lax
from jax.experimental import pallas as pl
from jax.experimental.pallas import tpu as pltpu
from jax.experimental.pallas import tpu_sc as plsc

N = 10000
E = 320000
D_IN = 128
D_HID = 32
D_OUT = 16

NC = 2
NS = 16
NW = NC * NS

CH = 128
CPT = -(-E // (NW * CH))
EP = NW * CPT * CH

NP = 10240
RPS = NP // NS

_mesh = plsc.VectorSubcoreMesh(core_axis_name="c", subcore_axis_name="s")


@functools.partial(
    pl.kernel,
    out_type=jax.ShapeDtypeStruct((NC, NP), jnp.float32),
    mesh=_mesh,
    scratch_types=[
        pltpu.VMEM((CPT, CH), jnp.int32),
        pltpu.VMEM((CH,), jnp.float32),
        pltpu.VMEM_SHARED((NP,), jnp.float32),
        pltpu.SemaphoreType.DMA,
    ],
)
def _sc_deg(z_hbm, dst_hbm, out_hbm, dst_v, ones_v, acc_sh, sem):
    c = lax.axis_index("c")
    s = lax.axis_index("s")
    wid = c * NS + s
    row0 = s * RPS
    pltpu.sync_copy(dst_hbm.at[wid], dst_v)
    for k in range(CH // 16):
        ones_v[pl.ds(k * 16, 16)] = jnp.full((16,), 1.0, jnp.float32)
    pltpu.sync_copy(z_hbm.at[pl.ds(row0, RPS)], acc_sh.at[pl.ds(row0, RPS)])
    plsc.subcore_barrier()

    def body(j, carry):
        pltpu.sync_copy(ones_v, acc_sh.at[dst_v.at[j]], add=True)
        return carry

    lax.fori_loop(0, CPT, body, 0)
    plsc.subcore_barrier()
    pltpu.sync_copy(acc_sh.at[pl.ds(row0, RPS)], out_hbm.at[c, pl.ds(row0, RPS)])


LOOK = 2
NBUF = 2 * LOOK


@functools.partial(
    pl.kernel,
    out_type=jax.ShapeDtypeStruct((NC, NP, D_HID), jnp.float32),
    mesh=_mesh,
    scratch_types=[
        pltpu.VMEM((CPT, CH), jnp.int32),
        pltpu.VMEM((CPT, CH), jnp.int32),
        pltpu.VMEM((NBUF, CH, D_HID), jnp.float32),
        pltpu.VMEM_SHARED((NP, D_HID), jnp.float32),
        pltpu.SemaphoreType.DMA,
        pltpu.SemaphoreType.DMA,
    ],
    compiler_params=pltpu.CompilerParams(use_tc_tiling_on_sc=False),
)
def _sc_pass(g_hbm, src_hbm, dst_hbm, out_hbm, src_v, dst_v, rows,
             acc_sh, gsem, ssem):
    c = lax.axis_index("c")
    s = lax.axis_index("s")
    wid = c * NS + s
    row0 = s * RPS
    pltpu.sync_copy(src_hbm.at[wid], src_v)
    pltpu.sync_copy(dst_hbm.at[wid], dst_v)
    pltpu.sync_copy(g_hbm.at[pl.ds(row0, RPS)], acc_sh.at[pl.ds(row0, RPS)])
    plsc.subcore_barrier()

    def gwait():
        pltpu.make_async_copy(g_hbm.at[src_v.at[0]], rows.at[0], gsem).wait()

    def swait():
        pltpu.make_async_copy(rows.at[0], acc_sh.at[dst_v.at[0]], ssem).wait()

    for j in range(LOOK):
        pltpu.async_copy(g_hbm.at[src_v.at[j]], rows.at[j], gsem)

    def body(j, carry):
        @pl.when(j >= NBUF - LOOK)
        def _():
            swait()

        @pl.when(j + LOOK < CPT)
        def _():
            pltpu.async_copy(g_hbm.at[src_v.at[j + LOOK]],
                             rows.at[(j + LOOK) % NBUF], gsem)

        gwait()
        pltpu.async_copy(rows.at[j % NBUF], acc_sh.at[dst_v.at[j]], ssem,
                         add=True)
        return carry

    lax.fori_loop(0, CPT, body, 0)
    for _ in range(NBUF - LOOK):
        swait()
    plsc.subcore_barrier()
    pltpu.sync_copy(acc_sh.at[pl.ds(row0, RPS)], out_hbm.at[c, pl.ds(row0, RPS)])


def _mm1_body(x_ref, w_ref, deg_ref, g_ref, dinv_ref):
    dinv = lax.rsqrt(deg_ref[0] + deg_ref[1] + 1.0)
    h = jnp.dot(x_ref[...], w_ref[...], preferred_element_type=jnp.float32)
    g_ref[...] = h * dinv
    dinv_ref[...] = dinv


_mm1 = pl.pallas_call(
    _mm1_body,
    out_shape=(
        jax.ShapeDtypeStruct((NP, D_HID), jnp.float32),
        jax.ShapeDtypeStruct((NP, 1), jnp.float32),
    ),
)


def _mid_body(b_ref, g1_ref, dinv_ref, bias_ref, g2_ref):
    agg = dinv_ref[...] * (b_ref[0] + b_ref[1] - g1_ref[...])
    h = jnp.maximum(agg + bias_ref[...], 0.0)
    g2_ref[...] = dinv_ref[...] * h


_mid = pl.pallas_call(
    _mid_body,
    out_shape=jax.ShapeDtypeStruct((NP, D_HID), jnp.float32),
)


def _fin_body(b_ref, g2_ref, dinv_ref, wmu_ref, bmu_ref, wls_ref, bls_ref,
              mu_ref, ls_ref):
    a = dinv_ref[...] * (b_ref[0] + b_ref[1] - g2_ref[...])
    mu_ref[...] = jnp.dot(a, wmu_ref[...], preferred_element_type=jnp.float32) + bmu_ref[...]
    ls_ref[...] = jnp.dot(a, wls_ref[...], preferred_element_type=jnp.float32) + bls_ref[...]


_fin = pl.pallas_call(
    _fin_body,
    out_shape=(
        jax.ShapeDtypeStruct((NP, D_OUT), jnp.float32),
        jax.ShapeDtypeStruct((NP, D_OUT), jnp.float32),
    ),
)


def kernel(x, edge_index, W1, b1, W_mu, b_mu, W_ls, b_ls):
    src = edge_index[0]
    dst = edge_index[1]
    pad = EP - E
    srcp = jnp.concatenate([src, jnp.zeros((pad,), src.dtype)]).reshape(NW, CPT, CH)
    dstp = jnp.concatenate([dst, jnp.full((pad,), N, dst.dtype)]).reshape(NW, CPT, CH)
    xp = jnp.concatenate([x, jnp.zeros((NP - N, D_IN), x.dtype)])
    z = jnp.zeros((NP,), jnp.float32)

    deg = _sc_deg(z, dstp).reshape(NC, NP, 1)
    g1, dinv = _mm1(xp, W1, deg)
    b1p = _sc_pass(g1, srcp, dstp)
    g2 = _mid(b1p, g1, dinv, b1.reshape(1, D_HID))
    b2p = _sc_pass(g2, srcp, dstp)
    mu, ls = _fin(b2p, g2, dinv, W_mu, b_mu.reshape(1, D_OUT),
                  W_ls, b_ls.reshape(1, D_OUT))
    return mu[:N], ls[:N]


# speedup vs baseline: 44.1802x; 1.0118x over previous
import functools

import jax
import jax.numpy as jnp
from jax import lax
from jax.experimental import pallas as pl
from jax.experimental.pallas import tpu as pltpu
from jax.experimental.pallas import tpu_sc as plsc

N = 10000
E = 320000
D_IN = 128
D_HID = 32
D_OUT = 16

NC = 2
NS = 16
NW = NC * NS

CH = 128
CPT = -(-E // (NW * CH))
EP = NW * CPT * CH

NP = 10240
RPS = NP // NS

_mesh = plsc.VectorSubcoreMesh(core_axis_name="c", subcore_axis_name="s")


@functools.partial(
    pl.kernel,
    out_type=jax.ShapeDtypeStruct((NC, NP), jnp.float32),
    mesh=_mesh,
    scratch_types=[
        pltpu.VMEM((CPT, CH), jnp.int32),
        pltpu.VMEM((CH,), jnp.float32),
        pltpu.VMEM_SHARED((NP,), jnp.float32),
        pltpu.SemaphoreType.DMA,
    ],
)
def _sc_deg(z_hbm, dst_hbm, out_hbm, dst_v, ones_v, acc_sh, sem):
    c = lax.axis_index("c")
    s = lax.axis_index("s")
    wid = c * NS + s
    row0 = s * RPS
    pltpu.sync_copy(dst_hbm.at[wid], dst_v)
    for k in range(CH // 16):
        ones_v[pl.ds(k * 16, 16)] = jnp.full((16,), 1.0, jnp.float32)
    pltpu.sync_copy(z_hbm.at[pl.ds(row0, RPS)], acc_sh.at[pl.ds(row0, RPS)])
    plsc.subcore_barrier()

    def body(j, carry):
        pltpu.sync_copy(ones_v, acc_sh.at[dst_v.at[j]], add=True)
        return carry

    lax.fori_loop(0, CPT, body, 0)
    plsc.subcore_barrier()
    pltpu.sync_copy(acc_sh.at[pl.ds(row0, RPS)], out_hbm.at[c, pl.ds(row0, RPS)])


LOOK = 4
NBUF = 2 * LOOK


@functools.partial(
    pl.kernel,
    out_type=jax.ShapeDtypeStruct((NC, NP, D_HID), jnp.float32),
    mesh=_mesh,
    scratch_types=[
        pltpu.VMEM((CPT, CH), jnp.int32),
        pltpu.VMEM((CPT, CH), jnp.int32),
        pltpu.VMEM((NBUF, CH, D_HID), jnp.float32),
        pltpu.VMEM_SHARED((NP, D_HID), jnp.float32),
        pltpu.SemaphoreType.DMA,
        pltpu.SemaphoreType.DMA,
    ],
    compiler_params=pltpu.CompilerParams(use_tc_tiling_on_sc=False),
)
def _sc_pass(g_hbm, src_hbm, dst_hbm, out_hbm, src_v, dst_v, rows,
             acc_sh, gsem, ssem):
    c = lax.axis_index("c")
    s = lax.axis_index("s")
    wid = c * NS + s
    row0 = s * RPS
    pltpu.sync_copy(src_hbm.at[wid], src_v)
    pltpu.sync_copy(dst_hbm.at[wid], dst_v)
    pltpu.sync_copy(g_hbm.at[pl.ds(row0, RPS)], acc_sh.at[pl.ds(row0, RPS)])
    plsc.subcore_barrier()

    def gwait():
        pltpu.make_async_copy(g_hbm.at[src_v.at[0]], rows.at[0], gsem).wait()

    def swait():
        pltpu.make_async_copy(rows.at[0], acc_sh.at[dst_v.at[0]], ssem).wait()

    for j in range(LOOK):
        pltpu.async_copy(g_hbm.at[src_v.at[j]], rows.at[j], gsem)

    def body(j, carry):
        @pl.when(j >= NBUF - LOOK)
        def _():
            swait()

        @pl.when(j + LOOK < CPT)
        def _():
            pltpu.async_copy(g_hbm.at[src_v.at[j + LOOK]],
                             rows.at[(j + LOOK) % NBUF], gsem)

        gwait()
        pltpu.async_copy(rows.at[j % NBUF], acc_sh.at[dst_v.at[j]], ssem,
                         add=True)
        return carry

    lax.fori_loop(0, CPT, body, 0)
    for _ in range(NBUF - LOOK):
        swait()
    plsc.subcore_barrier()
    pltpu.sync_copy(acc_sh.at[pl.ds(row0, RPS)], out_hbm.at[c, pl.ds(row0, RPS)])


def _mm1_body(x_ref, w_ref, deg_ref, g_ref, dinv_ref):
    dinv = lax.rsqrt(deg_ref[0] + deg_ref[1] + 1.0)
    h = jnp.dot(x_ref[...], w_ref[...], preferred_element_type=jnp.float32)
    g_ref[...] = h * dinv
    dinv_ref[...] = dinv


_mm1 = pl.pallas_call(
    _mm1_body,
    out_shape=(
        jax.ShapeDtypeStruct((NP, D_HID), jnp.float32),
        jax.ShapeDtypeStruct((NP, 1), jnp.float32),
    ),
)


def _mid_body(b_ref, g1_ref, dinv_ref, bias_ref, g2_ref):
    agg = dinv_ref[...] * (b_ref[0] + b_ref[1] - g1_ref[...])
    h = jnp.maximum(agg + bias_ref[...], 0.0)
    g2_ref[...] = dinv_ref[...] * h


_mid = pl.pallas_call(
    _mid_body,
    out_shape=jax.ShapeDtypeStruct((NP, D_HID), jnp.float32),
)


def _fin_body(b_ref, g2_ref, dinv_ref, wmu_ref, bmu_ref, wls_ref, bls_ref,
              mu_ref, ls_ref):
    a = dinv_ref[...] * (b_ref[0] + b_ref[1] - g2_ref[...])
    mu_ref[...] = jnp.dot(a, wmu_ref[...], preferred_element_type=jnp.float32) + bmu_ref[...]
    ls_ref[...] = jnp.dot(a, wls_ref[...], preferred_element_type=jnp.float32) + bls_ref[...]


_fin = pl.pallas_call(
    _fin_body,
    out_shape=(
        jax.ShapeDtypeStruct((NP, D_OUT), jnp.float32),
        jax.ShapeDtypeStruct((NP, D_OUT), jnp.float32),
    ),
)


def kernel(x, edge_index, W1, b1, W_mu, b_mu, W_ls, b_ls):
    src = edge_index[0]
    dst = edge_index[1]
    pad = EP - E
    srcp = jnp.concatenate([src, jnp.zeros((pad,), src.dtype)]).reshape(NW, CPT, CH)
    dstp = jnp.concatenate([dst, jnp.full((pad,), N, dst.dtype)]).reshape(NW, CPT, CH)
    xp = jnp.concatenate([x, jnp.zeros((NP - N, D_IN), x.dtype)])
    z = jnp.zeros((NP,), jnp.float32)

    deg = _sc_deg(z, dstp).reshape(NC, NP, 1)
    g1, dinv = _mm1(xp, W1, deg)
    b1p = _sc_pass(g1, srcp, dstp)
    g2 = _mid(b1p, g1, dinv, b1.reshape(1, D_HID))
    b2p = _sc_pass(g2, srcp, dstp)
    mu, ls = _fin(b2p, g2, dinv, W_mu, b_mu.reshape(1, D_OUT),
                  W_ls, b_ls.reshape(1, D_OUT))
    return mu[:N], ls[:N]


# speedup vs baseline: 59.8355x; 1.3544x over previous
import functools

import jax
import jax.numpy as jnp
from jax import lax
from jax.experimental import pallas as pl
from jax.experimental.pallas import tpu as pltpu
from jax.experimental.pallas import tpu_sc as plsc

N = 10000
E = 320000
D_IN = 128
D_HID = 32
D_OUT = 16

NC = 2
NS = 16
NW = NC * NS

CH = 128
CPT = -(-E // (NW * CH))
EP = NW * CPT * CH

NP = 10240
RPS = NP // NS

_mesh = plsc.VectorSubcoreMesh(core_axis_name="c", subcore_axis_name="s")


@functools.partial(
    pl.kernel,
    out_type=jax.ShapeDtypeStruct((NC, NP), jnp.float32),
    mesh=_mesh,
    scratch_types=[
        pltpu.VMEM((CPT, CH), jnp.int32),
        pltpu.VMEM((CH,), jnp.float32),
        pltpu.VMEM_SHARED((NP,), jnp.float32),
        pltpu.SemaphoreType.DMA,
    ],
)
def _sc_deg(z_hbm, dst_hbm, out_hbm, dst_v, ones_v, acc_sh, sem):
    c = lax.axis_index("c")
    s = lax.axis_index("s")
    wid = c * NS + s
    row0 = s * RPS
    pltpu.sync_copy(dst_hbm.at[wid], dst_v)
    for k in range(CH // 16):
        ones_v[pl.ds(k * 16, 16)] = jnp.full((16,), 1.0, jnp.float32)
    pltpu.sync_copy(z_hbm.at[pl.ds(row0, RPS)], acc_sh.at[pl.ds(row0, RPS)])
    plsc.subcore_barrier()

    def body(j, carry):
        pltpu.sync_copy(ones_v, acc_sh.at[dst_v.at[j]], add=True)
        return carry

    lax.fori_loop(0, CPT, body, 0)
    plsc.subcore_barrier()
    pltpu.sync_copy(acc_sh.at[pl.ds(row0, RPS)], out_hbm.at[c, pl.ds(row0, RPS)])


LOOK = 4
NBUF = 2 * LOOK


@functools.partial(
    pl.kernel,
    out_type=jax.ShapeDtypeStruct((NC, NP, D_HID), jnp.float32),
    mesh=_mesh,
    scratch_types=[
        pltpu.VMEM((CPT, CH), jnp.int32),
        pltpu.VMEM((CPT, CH), jnp.int32),
        pltpu.VMEM((NBUF, CH, D_HID), jnp.float32),
        pltpu.VMEM_SHARED((NP, D_HID), jnp.float32),
        pltpu.VMEM_SHARED((NP, D_HID), jnp.float32),
        pltpu.SemaphoreType.DMA,
        pltpu.SemaphoreType.DMA,
    ],
    compiler_params=pltpu.CompilerParams(use_tc_tiling_on_sc=False),
)
def _sc_pass(g_hbm, src_hbm, dst_hbm, out_hbm, src_v, dst_v, rows,
             acc_sh, g_sh, gsem, ssem):
    c = lax.axis_index("c")
    s = lax.axis_index("s")
    wid = c * NS + s
    row0 = s * RPS
    pltpu.sync_copy(src_hbm.at[wid], src_v)
    pltpu.sync_copy(dst_hbm.at[wid], dst_v)
    pltpu.sync_copy(g_hbm.at[pl.ds(row0, RPS)], acc_sh.at[pl.ds(row0, RPS)])
    pltpu.sync_copy(g_hbm.at[pl.ds(row0, RPS)], g_sh.at[pl.ds(row0, RPS)])
    plsc.subcore_barrier()

    def gwait():
        pltpu.make_async_copy(g_sh.at[src_v.at[0]], rows.at[0], gsem).wait()

    def swait():
        pltpu.make_async_copy(rows.at[0], acc_sh.at[dst_v.at[0]], ssem).wait()

    for j in range(LOOK):
        pltpu.async_copy(g_sh.at[src_v.at[j]], rows.at[j], gsem)

    def body(j, carry):
        @pl.when(j >= NBUF - LOOK)
        def _():
            swait()

        @pl.when(j + LOOK < CPT)
        def _():
            pltpu.async_copy(g_sh.at[src_v.at[j + LOOK]],
                             rows.at[(j + LOOK) % NBUF], gsem)

        gwait()
        pltpu.async_copy(rows.at[j % NBUF], acc_sh.at[dst_v.at[j]], ssem,
                         add=True)
        return carry

    lax.fori_loop(0, CPT, body, 0)
    for _ in range(NBUF - LOOK):
        swait()
    plsc.subcore_barrier()
    pltpu.sync_copy(acc_sh.at[pl.ds(row0, RPS)], out_hbm.at[c, pl.ds(row0, RPS)])


def _mm1_body(x_ref, w_ref, deg_ref, g_ref, dinv_ref):
    dinv = lax.rsqrt(deg_ref[0] + deg_ref[1] + 1.0)
    h = jnp.dot(x_ref[...], w_ref[...], preferred_element_type=jnp.float32)
    g_ref[...] = h * dinv
    dinv_ref[...] = dinv


_mm1 = pl.pallas_call(
    _mm1_body,
    out_shape=(
        jax.ShapeDtypeStruct((NP, D_HID), jnp.float32),
        jax.ShapeDtypeStruct((NP, 1), jnp.float32),
    ),
)


def _mid_body(b_ref, g1_ref, dinv_ref, bias_ref, g2_ref):
    agg = dinv_ref[...] * (b_ref[0] + b_ref[1] - g1_ref[...])
    h = jnp.maximum(agg + bias_ref[...], 0.0)
    g2_ref[...] = dinv_ref[...] * h


_mid = pl.pallas_call(
    _mid_body,
    out_shape=jax.ShapeDtypeStruct((NP, D_HID), jnp.float32),
)


def _fin_body(b_ref, g2_ref, dinv_ref, wmu_ref, bmu_ref, wls_ref, bls_ref,
              mu_ref, ls_ref):
    a = dinv_ref[...] * (b_ref[0] + b_ref[1] - g2_ref[...])
    mu_ref[...] = jnp.dot(a, wmu_ref[...], preferred_element_type=jnp.float32) + bmu_ref[...]
    ls_ref[...] = jnp.dot(a, wls_ref[...], preferred_element_type=jnp.float32) + bls_ref[...]


_fin = pl.pallas_call(
    _fin_body,
    out_shape=(
        jax.ShapeDtypeStruct((NP, D_OUT), jnp.float32),
        jax.ShapeDtypeStruct((NP, D_OUT), jnp.float32),
    ),
)


def kernel(x, edge_index, W1, b1, W_mu, b_mu, W_ls, b_ls):
    src = edge_index[0]
    dst = edge_index[1]
    pad = EP - E
    srcp = jnp.concatenate([src, jnp.zeros((pad,), src.dtype)]).reshape(NW, CPT, CH)
    dstp = jnp.concatenate([dst, jnp.full((pad,), N, dst.dtype)]).reshape(NW, CPT, CH)
    xp = jnp.concatenate([x, jnp.zeros((NP - N, D_IN), x.dtype)])
    z = jnp.zeros((NP,), jnp.float32)

    deg = _sc_deg(z, dstp).reshape(NC, NP, 1)
    g1, dinv = _mm1(xp, W1, deg)
    b1p = _sc_pass(g1, srcp, dstp)
    g2 = _mid(b1p, g1, dinv, b1.reshape(1, D_HID))
    b2p = _sc_pass(g2, srcp, dstp)
    mu, ls = _fin(b2p, g2, dinv, W_mu, b_mu.reshape(1, D_OUT),
                  W_ls, b_ls.reshape(1, D_OUT))
    return mu[:N], ls[:N]


# speedup vs baseline: 63.0000x; 1.0529x over previous
import functools

import jax
import jax.numpy as jnp
from jax import lax
from jax.experimental import pallas as pl
from jax.experimental.pallas import tpu as pltpu
from jax.experimental.pallas import tpu_sc as plsc

N = 10000
E = 320000
D_IN = 128
D_HID = 32
D_OUT = 16

NC = 2
NS = 16
NW = NC * NS

CH = 128
NCH = E // CH
CPT = -(-NCH // NW)

NP = 10240
RPS = NP // NS

_mesh = plsc.VectorSubcoreMesh(core_axis_name="c", subcore_axis_name="s")


def _chunk_window(wid):
    first = wid * CPT
    chunk0 = jnp.minimum(first, NCH - CPT)
    j0 = first - chunk0
    return chunk0, j0


@functools.partial(
    pl.kernel,
    out_type=jax.ShapeDtypeStruct((NC, NP), jnp.float32),
    mesh=_mesh,
    scratch_types=[
        pltpu.VMEM((CPT, CH), jnp.int32),
        pltpu.VMEM((CH,), jnp.float32),
        pltpu.VMEM_SHARED((NP,), jnp.float32),
        pltpu.SemaphoreType.DMA,
    ],
    compiler_params=pltpu.CompilerParams(use_tc_tiling_on_sc=False),
)
def _sc_deg(z_hbm, edges_hbm, out_hbm, dst_v, ones_v, acc_sh, sem):
    c = lax.axis_index("c")
    s = lax.axis_index("s")
    wid = c * NS + s
    row0 = s * RPS
    chunk0, j0 = _chunk_window(wid)
    pltpu.sync_copy(edges_hbm.at[1, pl.ds(chunk0, CPT)], dst_v)
    for k in range(CH // 16):
        ones_v[pl.ds(k * 16, 16)] = jnp.full((16,), 1.0, jnp.float32)
    pltpu.sync_copy(z_hbm.at[pl.ds(row0, RPS)], acc_sh.at[pl.ds(row0, RPS)])
    plsc.subcore_barrier()

    def body(j, carry):
        pltpu.sync_copy(ones_v, acc_sh.at[dst_v.at[j]], add=True)
        return carry

    lax.fori_loop(j0, CPT, body, 0)
    plsc.subcore_barrier()
    pltpu.sync_copy(acc_sh.at[pl.ds(row0, RPS)], out_hbm.at[c, pl.ds(row0, RPS)])


LOOK = 4
NBUF = 2 * LOOK


@functools.partial(
    pl.kernel,
    out_type=jax.ShapeDtypeStruct((NC, NP, D_HID), jnp.float32),
    mesh=_mesh,
    scratch_types=[
        pltpu.VMEM((CPT, CH), jnp.int32),
        pltpu.VMEM((CPT, CH), jnp.int32),
        pltpu.VMEM((NBUF, CH, D_HID), jnp.float32),
        pltpu.VMEM_SHARED((NP, D_HID), jnp.float32),
        pltpu.VMEM_SHARED((NP, D_HID), jnp.float32),
        pltpu.SemaphoreType.DMA,
        pltpu.SemaphoreType.DMA,
    ],
    compiler_params=pltpu.CompilerParams(use_tc_tiling_on_sc=False),
)
def _sc_pass(g_hbm, edges_hbm, out_hbm, src_v, dst_v, rows,
             acc_sh, g_sh, gsem, ssem):
    c = lax.axis_index("c")
    s = lax.axis_index("s")
    wid = c * NS + s
    row0 = s * RPS
    chunk0, j0 = _chunk_window(wid)
    nch = CPT - j0
    pltpu.sync_copy(edges_hbm.at[0, pl.ds(chunk0, CPT)], src_v)
    pltpu.sync_copy(edges_hbm.at[1, pl.ds(chunk0, CPT)], dst_v)
    pltpu.sync_copy(g_hbm.at[pl.ds(row0, RPS)], acc_sh.at[pl.ds(row0, RPS)])
    pltpu.sync_copy(g_hbm.at[pl.ds(row0, RPS)], g_sh.at[pl.ds(row0, RPS)])
    plsc.subcore_barrier()

    def gwait():
        pltpu.make_async_copy(g_sh.at[src_v.at[0]], rows.at[0], gsem).wait()

    def swait():
        pltpu.make_async_copy(rows.at[0], acc_sh.at[dst_v.at[0]], ssem).wait()

    for k in range(LOOK):
        pltpu.async_copy(g_sh.at[src_v.at[j0 + k]], rows.at[k], gsem)

    def body(k, carry):
        @pl.when(k >= NBUF - LOOK)
        def _():
            swait()

        @pl.when(k + LOOK < nch)
        def _():
            pltpu.async_copy(g_sh.at[src_v.at[j0 + k + LOOK]],
                             rows.at[(k + LOOK) % NBUF], gsem)

        gwait()
        pltpu.async_copy(rows.at[k % NBUF], acc_sh.at[dst_v.at[j0 + k]], ssem,
                         add=True)
        return carry

    lax.fori_loop(0, nch, body, 0)
    for _ in range(NBUF - LOOK):
        swait()
    plsc.subcore_barrier()
    pltpu.sync_copy(acc_sh.at[pl.ds(row0, RPS)], out_hbm.at[c, pl.ds(row0, RPS)])


def _mm1_body(x_ref, w_ref, deg_ref, g_ref, dinv_ref):
    dinv = lax.rsqrt(deg_ref[0] + deg_ref[1] + 1.0)
    h = jnp.dot(x_ref[...], w_ref[...], preferred_element_type=jnp.float32)
    g_ref[pl.ds(0, N)] = h * dinv[:N]
    g_ref[pl.ds(N, NP - N)] = jnp.zeros((NP - N, D_HID), jnp.float32)
    dinv_ref[...] = dinv


_mm1 = pl.pallas_call(
    _mm1_body,
    out_shape=(
        jax.ShapeDtypeStruct((NP, D_HID), jnp.float32),
        jax.ShapeDtypeStruct((NP, 1), jnp.float32),
    ),
)


def _mid_body(b_ref, g1_ref, dinv_ref, bias_ref, g2_ref):
    agg = dinv_ref[...] * (b_ref[0] + b_ref[1] - g1_ref[...])
    h = jnp.maximum(agg + bias_ref[...], 0.0)
    g2_ref[...] = dinv_ref[...] * h


_mid = pl.pallas_call(
    _mid_body,
    out_shape=jax.ShapeDtypeStruct((NP, D_HID), jnp.float32),
)


def _fin_body(b_ref, g2_ref, dinv_ref, wmu_ref, bmu_ref, wls_ref, bls_ref,
              mu_ref, ls_ref):
    a = dinv_ref[...] * (b_ref[0] + b_ref[1] - g2_ref[...])
    mu_ref[...] = jnp.dot(a, wmu_ref[...], preferred_element_type=jnp.float32) + bmu_ref[...]
    ls_ref[...] = jnp.dot(a, wls_ref[...], preferred_element_type=jnp.float32) + bls_ref[...]


_fin = pl.pallas_call(
    _fin_body,
    out_shape=(
        jax.ShapeDtypeStruct((NP, D_OUT), jnp.float32),
        jax.ShapeDtypeStruct((NP, D_OUT), jnp.float32),
    ),
)


def kernel(x, edge_index, W1, b1, W_mu, b_mu, W_ls, b_ls):
    e3 = edge_index.reshape(2, NCH, CH)
    z = jnp.zeros((NP,), jnp.float32)

    deg = _sc_deg(z, e3).reshape(NC, NP, 1)
    g1, dinv = _mm1(x, W1, deg)
    b1p = _sc_pass(g1, e3)
    g2 = _mid(b1p, g1, dinv, b1.reshape(1, D_HID))
    b2p = _sc_pass(g2, e3)
    mu, ls = _fin(b2p, g2, dinv, W_mu, b_mu.reshape(1, D_OUT),
                  W_ls, b_ls.reshape(1, D_OUT))
    return mu[:N], ls[:N]
